# baseline (speedup 1.0000x reference)
import jax
import jax.numpy as jnp
from jax.experimental import pallas as pl
from jax.experimental.pallas import tpu as pltpu

_G = 8


def _dma_kernel(order_ref, x_hbm, o_hbm, *sems):
    copies = []
    for i in range(_G):
        src = order_ref[i]
        copies.append(
            pltpu.make_async_copy(x_hbm.at[:, src], o_hbm.at[:, i], sems[i])
        )
    for c in copies:
        c.start()
    for c in copies:
        c.wait()


def kernel(x):
    N, C, H, W = x.shape
    g = _G
    perm = jax.random.permutation(jax.random.key(42), g - 1)
    order = jnp.concatenate(
        [perm, jnp.array([g - 1], dtype=perm.dtype)], axis=0
    ).astype(jnp.int32)
    xr = x.reshape(N, g, (C // g) * H * W)
    grid_spec = pltpu.PrefetchScalarGridSpec(
        num_scalar_prefetch=1,
        grid=(1,),
        in_specs=[pl.BlockSpec(memory_space=pl.ANY)],
        out_specs=pl.BlockSpec(memory_space=pl.ANY),
        scratch_shapes=[pltpu.SemaphoreType.DMA] * g,
    )
    out = pl.pallas_call(
        _dma_kernel,
        grid_spec=grid_spec,
        out_shape=jax.ShapeDtypeStruct(xr.shape, x.dtype),
    )(order, xr)
    return out.reshape(N, C, H, W)


# speedup vs baseline: 17.9749x; 17.9749x over previous
import jax
import jax.numpy as jnp
from jax.experimental import pallas as pl
from jax.experimental.pallas import tpu as pltpu

_G = 8
_BLK = 50176


def _copy_kernel(order_ref, x_ref, o_ref):
    o_ref[...] = x_ref[...]


def kernel(x):
    N, C, H, W = x.shape
    g = _G
    perm = jax.random.permutation(jax.random.key(42), g - 1)
    order = jnp.concatenate(
        [perm, jnp.array([g - 1], dtype=perm.dtype)], axis=0
    ).astype(jnp.int32)
    cg = C // g
    hw = H * W
    xr = x.reshape(N, g, cg, hw)
    blk = _BLK
    nj = hw // blk
    grid_spec = pltpu.PrefetchScalarGridSpec(
        num_scalar_prefetch=1,
        grid=(N, g, nj),
        in_specs=[
            pl.BlockSpec((1, 1, cg, blk), lambda n, i, j, order_ref: (n, order_ref[i], 0, j))
        ],
        out_specs=pl.BlockSpec((1, 1, cg, blk), lambda n, i, j, order_ref: (n, i, 0, j)),
    )
    out = pl.pallas_call(
        _copy_kernel,
        grid_spec=grid_spec,
        out_shape=jax.ShapeDtypeStruct((N, g, cg, hw), x.dtype),
    )(order, xr)
    return out.reshape(N, C, H, W)


# speedup vs baseline: 28.4887x; 1.5849x over previous
import jax
import jax.numpy as jnp
import numpy as np
from jax.experimental import pallas as pl
from jax.experimental.pallas import tpu as pltpu

_G = 8


def _copy_kernel(order_ref, x_ref, o_ref):
    o_ref[...] = x_ref[...]


def kernel(x):
    N, C, H, W = x.shape
    g = _G
    cg = C // g
    perm = jax.random.permutation(jax.random.key(42), g - 1)
    order = jnp.concatenate(
        [perm, jnp.array([g - 1], dtype=perm.dtype)], axis=0
    ).astype(jnp.int32)
    grid_spec = pltpu.PrefetchScalarGridSpec(
        num_scalar_prefetch=1,
        grid=(N, g),
        in_specs=[
            pl.BlockSpec((1, cg, H, W), lambda n, i, order_ref: (n, order_ref[i], 0, 0))
        ],
        out_specs=pl.BlockSpec((1, cg, H, W), lambda n, i, order_ref: (n, i, 0, 0)),
    )
    return pl.pallas_call(
        _copy_kernel,
        grid_spec=grid_spec,
        out_shape=jax.ShapeDtypeStruct((N, C, H, W), x.dtype),
    )(order, x)
